# initial kernel scaffold (unmeasured)
import jax
import jax.numpy as jnp
from jax import lax
from jax.experimental import pallas as pl
from jax.experimental.pallas import tpu as pltpu

B, Sq, D, Hq, Dh = 4, 256, 1024, 8, 128
NBH = B * Hq
SCALE = 0.08838834764831843


def kernel(x, Wq, Wo, K_ext, V_ext):
    xb = x.astype(jnp.bfloat16)
    wq = Wq.astype(jnp.bfloat16)
    wo = Wo.astype(jnp.bfloat16)
    kt = jnp.transpose(K_ext.astype(jnp.bfloat16), (0, 2, 1, 3))
    vt = jnp.transpose(V_ext.astype(jnp.bfloat16), (0, 2, 1, 3))

    def body(x_ref, wq_ref, wo_ref, k_ref, v_ref, out_ref,
             o_scr, ml_scr, o_recv, ml_recv, send_sems, recv_sems):
        my = lax.axis_index("i")
        p1 = my ^ 1
        p2 = 3 - my

        bar = pltpu.get_barrier_semaphore()
        for nbr in (p1, p2):
            pl.semaphore_signal(bar, inc=1, device_id=(nbr,),
                                device_id_type=pl.DeviceIdType.MESH)
        pl.semaphore_wait(bar, 2)

        wq_v = wq_ref[...]
        for b in range(B):
            qb = lax.dot_general(x_ref[b], wq_v, (((1,), (0,)), ((), ())),
                                 preferred_element_type=jnp.float32)
            qb = (qb * SCALE).astype(jnp.bfloat16)
            for h in range(Hq):
                col = b * Hq + h
                qh = qb[:, h * Dh:(h + 1) * Dh]
                s = lax.dot_general(qh, k_ref[b, h], (((1,), (1,)), ((), ())),
                                    preferred_element_type=jnp.float32)
                m = jnp.max(s, axis=1, keepdims=True)
                p = jnp.exp(s - m)
                l = jnp.sum(p, axis=1, keepdims=True)
                o = lax.dot_general(p.astype(jnp.bfloat16), v_ref[b, h],
                                    (((1,), (0,)), ((), ())),
                                    preferred_element_type=jnp.float32)
                o_scr[b, :, h * Dh:(h + 1) * Dh] = o
                ml_scr[:, col:col + 1] = m
                ml_scr[:, NBH + col:NBH + col + 1] = l

        for stage in range(2):
            partner = p1 if stage == 0 else p2
            rdma_o = pltpu.make_async_remote_copy(
                src_ref=o_scr, dst_ref=o_recv.at[stage],
                send_sem=send_sems.at[2 * stage], recv_sem=recv_sems.at[2 * stage],
                device_id=(partner,), device_id_type=pl.DeviceIdType.MESH)
            rdma_ml = pltpu.make_async_remote_copy(
                src_ref=ml_scr, dst_ref=ml_recv.at[stage],
                send_sem=send_sems.at[2 * stage + 1],
                recv_sem=recv_sems.at[2 * stage + 1],
                device_id=(partner,), device_id_type=pl.DeviceIdType.MESH)
            rdma_o.start()
            rdma_ml.start()
            rdma_o.wait()
            rdma_ml.wait()

            ml_a = ml_scr[...]
            ml_b = ml_recv[stage]
            m_new = jnp.maximum(ml_a[:, :NBH], ml_b[:, :NBH])
            a_a = jnp.exp(ml_a[:, :NBH] - m_new)
            a_b = jnp.exp(ml_b[:, :NBH] - m_new)
            ml_scr[:, :NBH] = m_new
            ml_scr[:, NBH:] = ml_a[:, NBH:] * a_a + ml_b[:, NBH:] * a_b
            for b in range(B):
                for h in range(Hq):
                    col = b * Hq + h
                    cols = pl.ds(h * Dh, Dh)
                    o_scr[b, :, cols] = (o_scr[b, :, cols] * a_a[:, col:col + 1]
                                         + o_recv[stage, b, :, cols] * a_b[:, col:col + 1])

        linv = 1.0 / ml_scr[:, NBH:]
        wo_v = wo_ref[...]
        for b in range(B):
            for h in range(Hq):
                col = b * Hq + h
                cols = pl.ds(h * Dh, Dh)
                o_scr[b, :, cols] = o_scr[b, :, cols] * linv[:, col:col + 1]
            out_ref[b] = lax.dot_general(o_scr[b].astype(jnp.bfloat16), wo_v,
                                         (((1,), (0,)), ((), ())),
                                         preferred_element_type=jnp.float32)

    return pl.pallas_call(
        body,
        out_shape=jax.ShapeDtypeStruct((B, Sq, D), jnp.float32),
        in_specs=[pl.BlockSpec(memory_space=pltpu.VMEM)] * 5,
        out_specs=pl.BlockSpec(memory_space=pltpu.VMEM),
        scratch_shapes=[
            pltpu.VMEM((B, Sq, D), jnp.float32),
            pltpu.VMEM((Sq, 2 * NBH), jnp.float32),
            pltpu.VMEM((2, B, Sq, D), jnp.float32),
            pltpu.VMEM((2, Sq, 2 * NBH), jnp.float32),
            pltpu.SemaphoreType.DMA((4,)),
            pltpu.SemaphoreType.DMA((4,)),
        ],
        compiler_params=pltpu.CompilerParams(collective_id=0),
    )(xb, wq, wo, kt, vt)


# baseline (device time: 180081 ns/iter reference)
import jax
import jax.numpy as jnp
from jax import lax
from jax.experimental import pallas as pl
from jax.experimental.pallas import tpu as pltpu

B, Sq, D, Hq, Dh = 4, 256, 1024, 8, 128
SCALE = 0.08838834764831843


def kernel(x, Wq, Wo, K_ext, V_ext):
    xb = x.astype(jnp.bfloat16)
    wq = Wq.astype(jnp.bfloat16)
    wo = Wo.astype(jnp.bfloat16)
    kt = jnp.transpose(K_ext.astype(jnp.bfloat16), (0, 2, 1, 3))
    vt = jnp.transpose(V_ext.astype(jnp.bfloat16), (0, 2, 1, 3))

    def body(x_ref, wq_ref, wo_ref, k_ref, v_ref, out_ref,
             q_scr, o_scr, ml_scr, a_scr, o_recv, ml_recv, send_sems, recv_sems):
        my = lax.axis_index("i")
        p1 = my ^ 1
        p2 = 3 - my

        bar = pltpu.get_barrier_semaphore()
        for nbr in (p1, p2):
            pl.semaphore_signal(bar, inc=1, device_id=(nbr,),
                                device_id_type=pl.DeviceIdType.MESH)
        pl.semaphore_wait(bar, 2)

        def attn_body(b, c):
            qb = lax.dot_general(x_ref[b], wq_ref[...], (((1,), (0,)), ((), ())),
                                 preferred_element_type=jnp.float32)
            q_scr[b] = (qb * SCALE).astype(jnp.bfloat16)
            for h in range(Hq):
                qh = q_scr[b, :, h * Dh:(h + 1) * Dh]
                s = lax.dot_general(qh, k_ref[b, h], (((1,), (1,)), ((), ())),
                                    preferred_element_type=jnp.float32)
                m = jnp.max(s, axis=1, keepdims=True)
                p = jnp.exp(s - m)
                l = jnp.sum(p, axis=1, keepdims=True)
                o = lax.dot_general(p.astype(jnp.bfloat16), v_ref[b, h],
                                    (((1,), (0,)), ((), ())),
                                    preferred_element_type=jnp.float32)
                o_scr[b, :, h * Dh:(h + 1) * Dh] = o
                ml_scr[b, :, h:h + 1] = m
                ml_scr[b, :, Hq + h:Hq + h + 1] = l
            return c
        lax.fori_loop(0, B, attn_body, 0)

        for stage in range(2):
            partner = p1 if stage == 0 else p2
            rdma_o = pltpu.make_async_remote_copy(
                src_ref=o_scr, dst_ref=o_recv.at[stage],
                send_sem=send_sems.at[2 * stage], recv_sem=recv_sems.at[2 * stage],
                device_id=(partner,), device_id_type=pl.DeviceIdType.MESH)
            rdma_ml = pltpu.make_async_remote_copy(
                src_ref=ml_scr, dst_ref=ml_recv.at[stage],
                send_sem=send_sems.at[2 * stage + 1],
                recv_sem=recv_sems.at[2 * stage + 1],
                device_id=(partner,), device_id_type=pl.DeviceIdType.MESH)
            rdma_o.start()
            rdma_ml.start()
            rdma_o.wait()
            rdma_ml.wait()

            ml_a = ml_scr[...]
            ml_b = ml_recv[stage]
            m_new = jnp.maximum(ml_a[:, :, :Hq], ml_b[:, :, :Hq])
            a_a = jnp.exp(ml_a[:, :, :Hq] - m_new)
            a_b = jnp.exp(ml_b[:, :, :Hq] - m_new)
            ml_scr[:, :, :Hq] = m_new
            ml_scr[:, :, Hq:] = ml_a[:, :, Hq:] * a_a + ml_b[:, :, Hq:] * a_b
            a_scr[0] = a_a
            a_scr[1] = a_b

            def merge_body(b, c, stage=stage):
                for h in range(Hq):
                    cols = slice(h * Dh, (h + 1) * Dh)
                    o_scr[b, :, cols] = (
                        o_scr[b, :, cols] * a_scr[0, b, :, h:h + 1]
                        + o_recv[stage, b, :, cols] * a_scr[1, b, :, h:h + 1])
                return c
            lax.fori_loop(0, B, merge_body, 0)

        a_scr[0] = 1.0 / ml_scr[:, :, Hq:]

        def out_body(b, c):
            for h in range(Hq):
                cols = slice(h * Dh, (h + 1) * Dh)
                o_scr[b, :, cols] = o_scr[b, :, cols] * a_scr[0, b, :, h:h + 1]
            out_ref[b] = lax.dot_general(o_scr[b].astype(jnp.bfloat16), wo_ref[...],
                                         (((1,), (0,)), ((), ())),
                                         preferred_element_type=jnp.float32)
            return c
        lax.fori_loop(0, B, out_body, 0)

    return pl.pallas_call(
        body,
        out_shape=jax.ShapeDtypeStruct((B, Sq, D), jnp.float32),
        in_specs=[pl.BlockSpec(memory_space=pltpu.VMEM)] * 5,
        out_specs=pl.BlockSpec(memory_space=pltpu.VMEM),
        scratch_shapes=[
            pltpu.VMEM((B, Sq, D), jnp.bfloat16),
            pltpu.VMEM((B, Sq, D), jnp.float32),
            pltpu.VMEM((B, Sq, 2 * Hq), jnp.float32),
            pltpu.VMEM((2, B, Sq, Hq), jnp.float32),
            pltpu.VMEM((2, B, Sq, D), jnp.float32),
            pltpu.VMEM((2, B, Sq, 2 * Hq), jnp.float32),
            pltpu.SemaphoreType.DMA((4,)),
            pltpu.SemaphoreType.DMA((4,)),
        ],
        compiler_params=pltpu.CompilerParams(
            collective_id=0, vmem_limit_bytes=100 * 1024 * 1024),
    )(xb, wq, wo, kt, vt)


# device time: 140862 ns/iter; 1.2784x vs baseline; 1.2784x over previous
import jax
import jax.numpy as jnp
from jax import lax
from jax.experimental import pallas as pl
from jax.experimental.pallas import tpu as pltpu

B, Sq, D, Hq, Dh = 4, 256, 1024, 8, 128
SCALE = 0.08838834764831843


def kernel(x, Wq, Wo, K_ext, V_ext):
    Skv = K_ext.shape[1]
    xb = x.astype(jnp.bfloat16)
    wq = Wq.astype(jnp.bfloat16)
    wo = Wo.astype(jnp.bfloat16)
    kt = jnp.reshape(K_ext.astype(jnp.bfloat16), (B, Skv, D))
    vt = jnp.reshape(V_ext.astype(jnp.bfloat16), (B, Skv, D))

    def body(x_ref, wq_ref, wo_ref, k_ref, v_ref, out_ref,
             q_scr, o_scr, o_send, ml_scr, a_scr, o_recv, ml_recv,
             send_sems, recv_sems):
        my = lax.axis_index("i")
        p1 = my ^ 1
        p2 = 3 - my

        bar = pltpu.get_barrier_semaphore()
        for nbr in (p1, p2):
            pl.semaphore_signal(bar, inc=1, device_id=(nbr,),
                                device_id_type=pl.DeviceIdType.MESH)
        pl.semaphore_wait(bar, 2)

        def attn_body(b, c):
            qb = lax.dot_general(x_ref[b], wq_ref[...], (((1,), (0,)), ((), ())),
                                 preferred_element_type=jnp.float32)
            q_scr[b] = (qb * SCALE).astype(jnp.bfloat16)
            for h in range(Hq):
                cols = slice(h * Dh, (h + 1) * Dh)
                qh = q_scr[b, :, cols]
                s = lax.dot_general(qh, k_ref[b, :, cols], (((1,), (1,)), ((), ())),
                                    preferred_element_type=jnp.float32)
                m = jnp.max(s, axis=1, keepdims=True)
                p = jnp.exp(s - m)
                l = jnp.sum(p, axis=1, keepdims=True)
                o = lax.dot_general(p.astype(jnp.bfloat16), v_ref[b, :, cols],
                                    (((1,), (0,)), ((), ())),
                                    preferred_element_type=jnp.float32)
                o_scr[b, :, cols] = o
                ml_scr[b, :, h:h + 1] = m
                ml_scr[b, :, Hq + h:Hq + h + 1] = l
            return c
        lax.fori_loop(0, B, attn_body, 0)

        for stage in range(2):
            partner = p1 if stage == 0 else p2
            o_send[...] = o_scr[...].astype(jnp.bfloat16)
            rdma_o = pltpu.make_async_remote_copy(
                src_ref=o_send, dst_ref=o_recv.at[stage],
                send_sem=send_sems.at[2 * stage], recv_sem=recv_sems.at[2 * stage],
                device_id=(partner,), device_id_type=pl.DeviceIdType.MESH)
            rdma_ml = pltpu.make_async_remote_copy(
                src_ref=ml_scr, dst_ref=ml_recv.at[stage],
                send_sem=send_sems.at[2 * stage + 1],
                recv_sem=recv_sems.at[2 * stage + 1],
                device_id=(partner,), device_id_type=pl.DeviceIdType.MESH)
            rdma_o.start()
            rdma_ml.start()
            rdma_o.wait()
            rdma_ml.wait()

            ml_a = ml_scr[...]
            ml_b = ml_recv[stage]
            m_new = jnp.maximum(ml_a[:, :, :Hq], ml_b[:, :, :Hq])
            a_a = jnp.exp(ml_a[:, :, :Hq] - m_new)
            a_b = jnp.exp(ml_b[:, :, :Hq] - m_new)
            ml_scr[:, :, :Hq] = m_new
            ml_scr[:, :, Hq:] = ml_a[:, :, Hq:] * a_a + ml_b[:, :, Hq:] * a_b
            a_scr[0] = a_a
            a_scr[1] = a_b

            def merge_body(b, c, stage=stage):
                for h in range(Hq):
                    cols = slice(h * Dh, (h + 1) * Dh)
                    o_scr[b, :, cols] = (
                        o_scr[b, :, cols] * a_scr[0, b, :, h:h + 1]
                        + o_recv[stage, b, :, cols].astype(jnp.float32)
                        * a_scr[1, b, :, h:h + 1])
                return c
            lax.fori_loop(0, B, merge_body, 0)

        a_scr[0] = 1.0 / ml_scr[:, :, Hq:]

        def out_body(b, c):
            for h in range(Hq):
                cols = slice(h * Dh, (h + 1) * Dh)
                o_scr[b, :, cols] = o_scr[b, :, cols] * a_scr[0, b, :, h:h + 1]
            out_ref[b] = lax.dot_general(o_scr[b].astype(jnp.bfloat16), wo_ref[...],
                                         (((1,), (0,)), ((), ())),
                                         preferred_element_type=jnp.float32)
            return c
        lax.fori_loop(0, B, out_body, 0)

    return pl.pallas_call(
        body,
        out_shape=jax.ShapeDtypeStruct((B, Sq, D), jnp.float32),
        in_specs=[pl.BlockSpec(memory_space=pltpu.VMEM)] * 5,
        out_specs=pl.BlockSpec(memory_space=pltpu.VMEM),
        scratch_shapes=[
            pltpu.VMEM((B, Sq, D), jnp.bfloat16),
            pltpu.VMEM((B, Sq, D), jnp.float32),
            pltpu.VMEM((B, Sq, D), jnp.bfloat16),
            pltpu.VMEM((B, Sq, 2 * Hq), jnp.float32),
            pltpu.VMEM((2, B, Sq, Hq), jnp.float32),
            pltpu.VMEM((2, B, Sq, D), jnp.bfloat16),
            pltpu.VMEM((2, B, Sq, 2 * Hq), jnp.float32),
            pltpu.SemaphoreType.DMA((4,)),
            pltpu.SemaphoreType.DMA((4,)),
        ],
        compiler_params=pltpu.CompilerParams(
            collective_id=0, vmem_limit_bytes=100 * 1024 * 1024),
    )(xb, wq, wo, kt, vt)
